# Initial kernel scaffold; baseline (speedup 1.0000x reference)
import jax
import jax.numpy as jnp
from jax.experimental import pallas as pl


def kernel(atom_x, bond_x, atom_edge_index, params):
    raise NotImplementedError("write your pallas kernel here")


# speedup vs baseline: 8.6760x; 8.6760x over previous
import functools

import jax
import jax.numpy as jnp
from jax import lax
from jax.experimental import pallas as pl
from jax.experimental.pallas import tpu as pltpu
from jax.experimental.pallas import tpu_sc as plsc

N = 10000
E = 320000
H = 128
ED = 16
NP = 10240
BN = 256
BE = 512

NC = 2
NS = 16
NW = NC * NS
EPW = E // NW
C = 80
NCH = EPW // C


def _leaky(x):
    return jnp.where(x >= 0, x, 0.01 * x)


def _elu(x):
    return jnp.where(x > 0, x, jnp.exp(jnp.minimum(x, 0.0)) - 1.0)


def _lin1_body(x_ref, w_ref, b_ref, o_ref):
    o_ref[...] = _leaky(
        jnp.dot(x_ref[...], w_ref[...], preferred_element_type=jnp.float32)
        + b_ref[...]
    )


def _tc_lin1(x, w_t, b):
    return pl.pallas_call(
        _lin1_body,
        grid=(NP // BN,),
        in_specs=[
            pl.BlockSpec((BN, H), lambda i: (i, 0)),
            pl.BlockSpec((H, H), lambda i: (0, 0)),
            pl.BlockSpec((1, H), lambda i: (0, 0)),
        ],
        out_specs=pl.BlockSpec((BN, H), lambda i: (i, 0)),
        out_shape=jax.ShapeDtypeStruct((NP, H), jnp.float32),
    )(x, w_t, b)


def _proj0_body(x_ref, wa_ref, wm_ref, ar_ref, xa_ref, xm_ref, arv_ref, am_ref):
    x = x_ref[...]
    xa_ref[...] = jnp.dot(x, wa_ref[...], preferred_element_type=jnp.float32)
    xm_ref[...] = jnp.dot(x, wm_ref[...], preferred_element_type=jnp.float32)
    arv = jnp.dot(x, ar_ref[...], preferred_element_type=jnp.float32)
    arv_ref[...] = arv

    @pl.when(pl.program_id(0) == 0)
    def _():
        am_ref[0, 0] = -jnp.inf

    am_ref[0, 0] = jnp.maximum(am_ref[0, 0], jnp.max(arv))


def _tc_proj0(x, wa_t, wm_t, att_r):
    return pl.pallas_call(
        _proj0_body,
        grid=(NP // BN,),
        in_specs=[
            pl.BlockSpec((BN, H), lambda i: (i, 0)),
            pl.BlockSpec((H, H), lambda i: (0, 0)),
            pl.BlockSpec((H, H), lambda i: (0, 0)),
            pl.BlockSpec((H, 1), lambda i: (0, 0)),
        ],
        out_specs=[
            pl.BlockSpec((BN, H), lambda i: (i, 0)),
            pl.BlockSpec((BN, H), lambda i: (i, 0)),
            pl.BlockSpec((BN, 1), lambda i: (i, 0)),
            pl.BlockSpec((1, 1), lambda i: (0, 0), memory_space=pltpu.SMEM),
        ],
        out_shape=[
            jax.ShapeDtypeStruct((NP, H), jnp.float32),
            jax.ShapeDtypeStruct((NP, H), jnp.float32),
            jax.ShapeDtypeStruct((NP, 1), jnp.float32),
            jax.ShapeDtypeStruct((1, 1), jnp.float32),
        ],
    )(x, wa_t, wm_t, att_r)


def _edge_q_body(xg_ref, bond_ref, wb_ref, al_ref, q_ref, qm_ref):
    t = _leaky(
        xg_ref[...]
        + jnp.dot(bond_ref[...], wb_ref[...], preferred_element_type=jnp.float32)
    )
    q = jnp.dot(t, al_ref[...], preferred_element_type=jnp.float32)
    q_ref[...] = q

    @pl.when(pl.program_id(0) == 0)
    def _():
        qm_ref[0, 0] = -jnp.inf

    qm_ref[0, 0] = jnp.maximum(qm_ref[0, 0], jnp.max(q))


def _tc_edge_q(xg, bond, wb_t, att_l):
    return pl.pallas_call(
        _edge_q_body,
        grid=(E // BE,),
        in_specs=[
            pl.BlockSpec((BE, H), lambda i: (i, 0)),
            pl.BlockSpec((BE, ED), lambda i: (i, 0)),
            pl.BlockSpec((ED, H), lambda i: (0, 0)),
            pl.BlockSpec((H, 1), lambda i: (0, 0)),
        ],
        out_specs=[
            pl.BlockSpec((BE, 1), lambda i: (i, 0)),
            pl.BlockSpec((1, 1), lambda i: (0, 0), memory_space=pltpu.SMEM),
        ],
        out_shape=[
            jax.ShapeDtypeStruct((E, 1), jnp.float32),
            jax.ShapeDtypeStruct((1, 1), jnp.float32),
        ],
    )(xg, bond, wb_t, att_l)


def _recip_body(parts_ref, r_ref):
    r_ref[...] = 1.0 / (
        jnp.sum(parts_ref[...], axis=0, keepdims=True) + 1e-16
    )


def _tc_recip(parts):
    return pl.pallas_call(
        _recip_body,
        in_specs=[pl.BlockSpec((NW, NP), lambda: (0, 0))],
        out_specs=pl.BlockSpec((1, NP), lambda: (0, 0)),
        out_shape=jax.ShapeDtypeStruct((1, NP), jnp.float32),
    )(parts)


def _gru_body(hp0_ref, hp1_ref, r_ref, gb_ref, x_ref, wih_ref, whh_ref,
              bih_ref, bhh_ref, o_ref):
    h = _elu(r_ref[...] * (hp0_ref[...] + hp1_ref[...]) + gb_ref[...])
    x = x_ref[...]
    gi = jnp.dot(h, wih_ref[...], preferred_element_type=jnp.float32) + bih_ref[...]
    gh = jnp.dot(x, whh_ref[...], preferred_element_type=jnp.float32) + bhh_ref[...]
    i_r, i_z, i_n = gi[:, :H], gi[:, H:2 * H], gi[:, 2 * H:]
    h_r, h_z, h_n = gh[:, :H], gh[:, H:2 * H], gh[:, 2 * H:]
    rr = jax.nn.sigmoid(i_r + h_r)
    zz = jax.nn.sigmoid(i_z + h_z)
    nn = jnp.tanh(i_n + rr * h_n)
    o_ref[...] = jax.nn.relu((1.0 - zz) * nn + zz * x)


def _tc_gru(hp0, hp1, r_col, gb, x, wih_t, whh_t, bih, bhh):
    return pl.pallas_call(
        _gru_body,
        grid=(NP // BN,),
        in_specs=[
            pl.BlockSpec((BN, H), lambda i: (i, 0)),
            pl.BlockSpec((BN, H), lambda i: (i, 0)),
            pl.BlockSpec((BN, 1), lambda i: (i, 0)),
            pl.BlockSpec((1, H), lambda i: (0, 0)),
            pl.BlockSpec((BN, H), lambda i: (i, 0)),
            pl.BlockSpec((H, 3 * H), lambda i: (0, 0)),
            pl.BlockSpec((H, 3 * H), lambda i: (0, 0)),
            pl.BlockSpec((1, 3 * H), lambda i: (0, 0)),
            pl.BlockSpec((1, 3 * H), lambda i: (0, 0)),
        ],
        out_specs=pl.BlockSpec((BN, H), lambda i: (i, 0)),
        out_shape=jax.ShapeDtypeStruct((NP, H), jnp.float32),
    )(hp0, hp1, r_col, gb, x, wih_t, whh_t, bih, bhh)


def _proj12_body(x_ref, w_ref, asrc_ref, adst_ref, xl_ref, s_ref, d_ref,
                 sm_ref, dm_ref):
    xl = jnp.dot(x_ref[...], w_ref[...], preferred_element_type=jnp.float32)
    xl_ref[...] = xl
    s = jnp.dot(xl, asrc_ref[...], preferred_element_type=jnp.float32)
    d = jnp.dot(xl, adst_ref[...], preferred_element_type=jnp.float32)
    s_ref[...] = s
    d_ref[...] = d

    @pl.when(pl.program_id(0) == 0)
    def _():
        sm_ref[0, 0] = -jnp.inf
        dm_ref[0, 0] = -jnp.inf

    sm_ref[0, 0] = jnp.maximum(sm_ref[0, 0], jnp.max(s))
    dm_ref[0, 0] = jnp.maximum(dm_ref[0, 0], jnp.max(d))


def _tc_proj12(x, w_t, att_src, att_dst):
    return pl.pallas_call(
        _proj12_body,
        grid=(NP // BN,),
        in_specs=[
            pl.BlockSpec((BN, H), lambda i: (i, 0)),
            pl.BlockSpec((H, H), lambda i: (0, 0)),
            pl.BlockSpec((H, 1), lambda i: (0, 0)),
            pl.BlockSpec((H, 1), lambda i: (0, 0)),
        ],
        out_specs=[
            pl.BlockSpec((BN, H), lambda i: (i, 0)),
            pl.BlockSpec((BN, 1), lambda i: (i, 0)),
            pl.BlockSpec((BN, 1), lambda i: (i, 0)),
            pl.BlockSpec((1, 1), lambda i: (0, 0), memory_space=pltpu.SMEM),
            pl.BlockSpec((1, 1), lambda i: (0, 0), memory_space=pltpu.SMEM),
        ],
        out_shape=[
            jax.ShapeDtypeStruct((NP, H), jnp.float32),
            jax.ShapeDtypeStruct((NP, 1), jnp.float32),
            jax.ShapeDtypeStruct((NP, 1), jnp.float32),
            jax.ShapeDtypeStruct((1, 1), jnp.float32),
            jax.ShapeDtypeStruct((1, 1), jnp.float32),
        ],
    )(x, w_t, att_src, att_dst)


def _final_body(x_ref, w_ref, b_ref, o_ref):
    o_ref[...] = (
        jnp.dot(x_ref[...], w_ref[...], preferred_element_type=jnp.float32)
        + b_ref[...]
    )


def _tc_final(x, w_t, b):
    return pl.pallas_call(
        _final_body,
        grid=(NP // BN,),
        in_specs=[
            pl.BlockSpec((BN, H), lambda i: (i, 0)),
            pl.BlockSpec((H, H), lambda i: (0, 0)),
            pl.BlockSpec((1, H), lambda i: (0, 0)),
        ],
        out_specs=pl.BlockSpec((BN, H), lambda i: (i, 0)),
        out_shape=jax.ShapeDtypeStruct((NP, H), jnp.float32),
    )(x, w_t, b)


_MESH = plsc.VectorSubcoreMesh(
    core_axis_name="c", subcore_axis_name="s", num_cores=NC, num_subcores=NS)
_SC_PARAMS = pltpu.CompilerParams(needs_layout_passes=False)


def _worker_id():
    return lax.axis_index("s") * NC + lax.axis_index("c")


def _sc_gather_rows(table, src3):
    @functools.partial(
        pl.kernel,
        out_type=jax.ShapeDtypeStruct((E, H), jnp.float32),
        mesh=_MESH,
        compiler_params=_SC_PARAMS,
        scratch_types=[
            pltpu.VMEM((NCH, C), jnp.int32),
            pltpu.VMEM((C, H), jnp.float32),
            pltpu.SemaphoreType.DMA,
        ],
    )
    def k(table_hbm, src_hbm, out_hbm, idx_v, rows_v, sem):
        wid = _worker_id()
        pltpu.sync_copy(src_hbm.at[wid], idx_v)
        base0 = wid * EPW

        @pl.loop(0, NCH)
        def _(j):
            pltpu.async_copy(table_hbm.at[idx_v.at[j]], rows_v, sem).wait()
            pltpu.sync_copy(rows_v, out_hbm.at[pl.ds(base0 + j * C, C)])

    return k(table, src3)


def _sc_edge_softmax(src3, dst3, mvec, s_nodes, d_nodes, q3):
    layer0 = q3 is not None
    ins = (src3, dst3, mvec, s_nodes, d_nodes) + ((q3,) if layer0 else ())

    @functools.partial(
        pl.kernel,
        out_type=[
            jax.ShapeDtypeStruct((NW, NCH, C), jnp.float32),
            jax.ShapeDtypeStruct((NW, NP), jnp.float32),
        ],
        mesh=_MESH,
        compiler_params=_SC_PARAMS,
        scratch_types=[
            pltpu.VMEM((NCH, C), jnp.int32),
            pltpu.VMEM((NCH, C), jnp.int32),
            pltpu.VMEM((16,), jnp.float32),
            pltpu.VMEM((NP,), jnp.float32),
            pltpu.VMEM((NP,), jnp.float32),
            pltpu.VMEM((NCH, C), jnp.float32),
            pltpu.VMEM((NP,), jnp.float32),
        ],
    )
    def k(*refs):
        if layer0:
            (src_hbm, dst_hbm, m_hbm, s_hbm, d_hbm, q_hbm,
             ex_hbm, parts_hbm,
             src_v, dst_v, m_v, s_v, d_v, ex_v, den_v) = refs
        else:
            (src_hbm, dst_hbm, m_hbm, s_hbm, d_hbm,
             ex_hbm, parts_hbm,
             src_v, dst_v, m_v, s_v, d_v, ex_v, den_v) = refs
        wid = _worker_id()
        pltpu.sync_copy(src_hbm.at[wid], src_v)
        pltpu.sync_copy(dst_hbm.at[wid], dst_v)
        pltpu.sync_copy(m_hbm, m_v)
        pltpu.sync_copy(s_hbm, s_v)
        if not layer0:
            pltpu.sync_copy(d_hbm, d_v)
        if layer0:
            pltpu.sync_copy(q_hbm.at[wid], ex_v)

        @pl.loop(0, NP // 16)
        def _(i):
            den_v[pl.ds(i * 16, 16)] = jnp.zeros((16,), jnp.float32)

        mv = m_v[...]

        @pl.loop(0, NCH)
        def _(j):
            for i in range(C // 16):
                sl = pl.ds(i * 16, 16)
                di = dst_v[j, sl]
                if layer0:
                    a = ex_v[j, sl] + plsc.load_gather(s_v, [di])
                else:
                    si = src_v[j, sl]
                    a = plsc.load_gather(s_v, [si]) + plsc.load_gather(d_v, [di])
                a = jnp.where(a >= 0, a, 0.01 * a)
                exv = jnp.exp(a - mv)
                ex_v[j, sl] = exv
                plsc.addupdate_scatter(den_v, [di], exv)

        pltpu.sync_copy(ex_v, ex_hbm.at[wid])
        pltpu.sync_copy(den_v, parts_hbm.at[wid])

    return k(*ins)


def _sc_aggregate(table, src3, dst3, ex3):
    RPT = NP // NS
    TD = 64

    @functools.partial(
        pl.kernel,
        out_type=jax.ShapeDtypeStruct((NC, NP, H), jnp.float32),
        mesh=_MESH,
        compiler_params=_SC_PARAMS,
        scratch_types=[
            pltpu.VMEM((C,), jnp.int32),
            pltpu.VMEM((C,), jnp.int32),
            pltpu.VMEM((NCH, C), jnp.float32),
            pltpu.VMEM((C, H), jnp.float32),
            pltpu.VMEM((TD, H), jnp.float32),
            pltpu.VMEM_SHARED((NP, H), jnp.float32),
            pltpu.SemaphoreType.DMA,
        ],
    )
    def k(table_hbm, src_hbm, dst_hbm, ex_hbm, out_hbm,
          src_v, dst_v, ex_v, rows_v, tmp_v, acc_sh, sem):
        cid = lax.axis_index("c")
        sid = lax.axis_index("s")
        wid = sid * NC + cid

        @pl.loop(0, TD)
        def _(i):
            for dd in range(H // 16):
                tmp_v[i, pl.ds(dd * 16, 16)] = jnp.zeros((16,), jnp.float32)

        @pl.loop(0, RPT // TD)
        def _(t):
            pltpu.sync_copy(tmp_v, acc_sh.at[pl.ds(sid * RPT + t * TD, TD)])

        plsc.subcore_barrier()

        pltpu.sync_copy(ex_hbm.at[wid], ex_v)

        cj = [jnp.full((16,), j, jnp.int32) for j in range(16)]

        @pl.loop(0, NCH)
        def _(j):
            pltpu.sync_copy(src_hbm.at[wid, j], src_v)
            pltpu.sync_copy(dst_hbm.at[wid, j], dst_v)
            pltpu.async_copy(table_hbm.at[src_v], rows_v, sem).wait()
            for i in range(C // 16):
                al16 = ex_v[j, pl.ds(i * 16, 16)]
                for jj in range(16):
                    av = jnp.take(al16, cj[jj])
                    e = i * 16 + jj
                    for dd in range(H // 16):
                        sl = pl.ds(dd * 16, 16)
                        rows_v[e, sl] = rows_v[e, sl] * av
            pltpu.sync_copy(rows_v, acc_sh.at[dst_v], add=True)

        plsc.subcore_barrier()

        @pl.loop(0, RPT // TD)
        def _(t):
            r0 = sid * RPT + t * TD
            pltpu.sync_copy(acc_sh.at[pl.ds(r0, TD)], tmp_v)
            pltpu.sync_copy(tmp_v, out_hbm.at[cid, pl.ds(r0, TD)])

    return k(table, src3, dst3, ex3)


def kernel(atom_x, bond_x, atom_edge_index, params):
    p = params
    src = atom_edge_index[0]
    dst = atom_edge_index[1]
    src3 = src.reshape(NW, NCH, C)
    dst3 = dst.reshape(NW, NCH, C)

    xpad = jnp.zeros((NP, H), jnp.float32).at[:N].set(atom_x)

    x = _tc_lin1(xpad, p['lin1_W'].T, p['lin1_b'].reshape(1, H))

    wa_t = p['gc_lin1_W'][:, :H].T
    wb_t = p['gc_lin1_W'][:, H:].T
    xa, xm, ar, armax = _tc_proj0(
        x, wa_t, p['gc_lin2_W'].T, p['gc_att_r'].reshape(H, 1))
    xg = _sc_gather_rows(xa, src3)
    q, qmax = _tc_edge_q(xg, bond_x, wb_t, p['gc_att_l'].reshape(H, 1))
    m0 = jnp.maximum(qmax[0, 0] + jnp.maximum(armax[0, 0], 0.0), 0.0)
    mvec = jnp.full((16,), m0, jnp.float32)
    ar_n = ar.reshape(NP)
    ex3, parts = _sc_edge_softmax(
        src3, dst3, mvec, ar_n, ar_n, q.reshape(NW, NCH, C))
    r = _tc_recip(parts).reshape(NP, 1)
    hp = _sc_aggregate(xm, src3, dst3, ex3)
    x = _tc_gru(hp[0], hp[1], r, p['gc_bias'].reshape(1, H), x,
                p['gru0_Wih'].T, p['gru0_Whh'].T,
                p['gru0_bih'].reshape(1, 3 * H), p['gru0_bhh'].reshape(1, 3 * H))

    for l in range(2):
        xl, s, d, smax, dmax = _tc_proj12(
            x, p[f'conv{l}_W'].T,
            p[f'conv{l}_att_src'].reshape(H, 1),
            p[f'conv{l}_att_dst'].reshape(H, 1))
        m = jnp.maximum(smax[0, 0] + dmax[0, 0], 0.0)
        mvec = jnp.full((16,), m, jnp.float32)
        ex3, parts = _sc_edge_softmax(
            src3, dst3, mvec, s.reshape(NP), d.reshape(NP), None)
        r = _tc_recip(parts).reshape(NP, 1)
        hp = _sc_aggregate(xl, src3, dst3, ex3)
        x = _tc_gru(hp[0], hp[1], r, p[f'conv{l}_bias'].reshape(1, H), x,
                    p[f'gru{l + 1}_Wih'].T, p[f'gru{l + 1}_Whh'].T,
                    p[f'gru{l + 1}_bih'].reshape(1, 3 * H),
                    p[f'gru{l + 1}_bhh'].reshape(1, 3 * H))

    out = _tc_final(x, p['lin2_W'].T, p['lin2_b'].reshape(1, H))
    return out[:N]
